# baseline (speedup 1.0000x reference)
import numpy as np
import jax
import jax.numpy as jnp
from jax.experimental import pallas as pl

_NUM_RES = 190
_CYT = np.arange(0, _NUM_RES, 4)
_NCYT = len(_CYT)
_HID = 10
_BLK = 512


def _body(d_ref, fit_ref, M_ref, m_ref, mcyt_ref, base_ref, slope_ref,
          b1_ref, w2a_ref, w2b_ref, b2_ref, out_ref):
    d = d_ref[...]
    G = jax.lax.dot_general(
        d, M_ref[...],
        dimension_numbers=(((1,), (0,)), ((), ())),
        preferred_element_type=jnp.float32,
        precision=jax.lax.Precision.HIGHEST)
    f = fit_ref[...]

    S2 = G[:, 0:1]
    Sxb = G[:, 1:2]
    Sxs = G[:, 2:3]

    h = jnp.maximum(G[:, 3:3 + _HID] + b1_ref[...], 0.0)
    b2 = b2_ref[...]
    p0 = jnp.sum(h * w2a_ref[...], axis=1, keepdims=True) + b2[:, 0:1]
    p1 = jnp.sum(h * w2b_ref[...], axis=1, keepdims=True) + b2[:, 1:2]

    a = base_ref[...] + slope_ref[...] * f
    mc = mcyt_ref[...]
    mx = jnp.maximum(jnp.maximum(a, mc), 0.0)
    lse = mx + jnp.log(jnp.exp(-mx) + jnp.exp(a - mx) + jnp.exp(mc - mx))
    sum_lse = jnp.sum(lse, axis=1, keepdims=True)

    m = m_ref[...]
    r = jax.lax.broadcasted_iota(jnp.int32, (1, _NUM_RES), 1)
    sp = jnp.maximum(m, 0.0) + jnp.log1p(jnp.exp(-jnp.abs(m)))
    c_non = jnp.sum(jnp.where((r % 4) != 0, sp, 0.0))

    logp = S2 + Sxb + Sxs * f - c_non - sum_lse
    out_ref[...] = jnp.concatenate([logp, p0, p1], axis=1)


def kernel(dharma_output, fitness, mutation_rates, baseline_edits,
           slope_edits, W1, b1, W2, b2):
    B = dharma_output.shape[0]
    F = _NUM_RES * 3
    d = dharma_output.reshape(B, F)

    M = jnp.zeros((F, 128), dtype=jnp.float32)
    M = M.at[3 * np.arange(_NUM_RES) + 2, 0].set(mutation_rates)
    M = M.at[3 * _CYT + 1, 1].set(baseline_edits)
    M = M.at[3 * _CYT + 1, 2].set(slope_edits)
    M = M.at[3 * _CYT + 1, 3:3 + _HID].set(W1)

    fit2 = fitness.reshape(B, 1)
    m2 = mutation_rates.reshape(1, _NUM_RES)
    mcyt2 = mutation_rates[_CYT].reshape(1, _NCYT)
    base2 = baseline_edits.reshape(1, _NCYT)
    slope2 = slope_edits.reshape(1, _NCYT)
    b1r = b1.reshape(1, _HID)
    w2a = W2[:, 0].reshape(1, _HID)
    w2b = W2[:, 1].reshape(1, _HID)
    b2r = b2.reshape(1, 2)

    grid = B // _BLK
    out = pl.pallas_call(
        _body,
        grid=(grid,),
        in_specs=[
            pl.BlockSpec((_BLK, F), lambda i: (i, 0)),
            pl.BlockSpec((_BLK, 1), lambda i: (i, 0)),
            pl.BlockSpec((F, 128), lambda i: (0, 0)),
            pl.BlockSpec((1, _NUM_RES), lambda i: (0, 0)),
            pl.BlockSpec((1, _NCYT), lambda i: (0, 0)),
            pl.BlockSpec((1, _NCYT), lambda i: (0, 0)),
            pl.BlockSpec((1, _NCYT), lambda i: (0, 0)),
            pl.BlockSpec((1, _HID), lambda i: (0, 0)),
            pl.BlockSpec((1, _HID), lambda i: (0, 0)),
            pl.BlockSpec((1, _HID), lambda i: (0, 0)),
            pl.BlockSpec((1, 2), lambda i: (0, 0)),
        ],
        out_specs=pl.BlockSpec((_BLK, 3), lambda i: (i, 0)),
        out_shape=jax.ShapeDtypeStruct((B, 3), jnp.float32),
    )(d, fit2, M, m2, mcyt2, base2, slope2, b1r, w2a, w2b, b2r)
    return out


# speedup vs baseline: 1.0575x; 1.0575x over previous
import numpy as np
import jax
import jax.numpy as jnp
from jax.experimental import pallas as pl

_NUM_RES = 190
_CYT = np.arange(0, _NUM_RES, 4)
_NCYT = len(_CYT)
_HID = 10
_BLK = 512


def _body(d_ref, fit_ref, M_ref, m_ref, mcyt_ref, base_ref, slope_ref,
          b1_ref, w2a_ref, w2b_ref, b2_ref, out_ref):
    d = d_ref[...].astype(jnp.bfloat16)
    G2 = jax.lax.dot_general(
        d, M_ref[...],
        dimension_numbers=(((1,), (0,)), ((), ())),
        preferred_element_type=jnp.float32)
    G = G2[:, 0:16] + G2[:, 64:80]
    f = fit_ref[...]

    S2 = G[:, 0:1]
    Sxb = G[:, 1:2]
    Sxs = G[:, 2:3]

    h = jnp.maximum(G[:, 3:3 + _HID] + b1_ref[...], 0.0)
    b2 = b2_ref[...]
    p0 = jnp.sum(h * w2a_ref[...], axis=1, keepdims=True) + b2[:, 0:1]
    p1 = jnp.sum(h * w2b_ref[...], axis=1, keepdims=True) + b2[:, 1:2]

    a = base_ref[...] + slope_ref[...] * f
    mc = mcyt_ref[...]
    mx = jnp.maximum(jnp.maximum(a, mc), 0.0)
    lse = mx + jnp.log(jnp.exp(-mx) + jnp.exp(a - mx) + jnp.exp(mc - mx))
    sum_lse = jnp.sum(lse, axis=1, keepdims=True)

    m = m_ref[...]
    r = jax.lax.broadcasted_iota(jnp.int32, (1, _NUM_RES), 1)
    sp = jnp.maximum(m, 0.0) + jnp.log1p(jnp.exp(-jnp.abs(m)))
    c_non = jnp.sum(jnp.where((r % 4) != 0, sp, 0.0))

    logp = S2 + Sxb + Sxs * f - c_non - sum_lse
    out_ref[...] = jnp.concatenate([logp, p0, p1], axis=1)


def kernel(dharma_output, fitness, mutation_rates, baseline_edits,
           slope_edits, W1, b1, W2, b2):
    B = dharma_output.shape[0]
    F = _NUM_RES * 3
    d = dharma_output.reshape(B, F)

    Mf = jnp.zeros((F, 16), dtype=jnp.float32)
    Mf = Mf.at[3 * np.arange(_NUM_RES) + 2, 0].set(mutation_rates)
    Mf = Mf.at[3 * _CYT + 1, 1].set(baseline_edits)
    Mf = Mf.at[3 * _CYT + 1, 2].set(slope_edits)
    Mf = Mf.at[3 * _CYT + 1, 3:3 + _HID].set(W1)
    M_hi = Mf.astype(jnp.bfloat16)
    M_lo = (Mf - M_hi.astype(jnp.float32)).astype(jnp.bfloat16)
    M = jnp.zeros((F, 128), dtype=jnp.bfloat16)
    M = M.at[:, 0:16].set(M_hi)
    M = M.at[:, 64:80].set(M_lo)

    fit2 = fitness.reshape(B, 1)
    m2 = mutation_rates.reshape(1, _NUM_RES)
    mcyt2 = mutation_rates[_CYT].reshape(1, _NCYT)
    base2 = baseline_edits.reshape(1, _NCYT)
    slope2 = slope_edits.reshape(1, _NCYT)
    b1r = b1.reshape(1, _HID)
    w2a = W2[:, 0].reshape(1, _HID)
    w2b = W2[:, 1].reshape(1, _HID)
    b2r = b2.reshape(1, 2)

    grid = B // _BLK
    out = pl.pallas_call(
        _body,
        grid=(grid,),
        in_specs=[
            pl.BlockSpec((_BLK, F), lambda i: (i, 0)),
            pl.BlockSpec((_BLK, 1), lambda i: (i, 0)),
            pl.BlockSpec((F, 128), lambda i: (0, 0)),
            pl.BlockSpec((1, _NUM_RES), lambda i: (0, 0)),
            pl.BlockSpec((1, _NCYT), lambda i: (0, 0)),
            pl.BlockSpec((1, _NCYT), lambda i: (0, 0)),
            pl.BlockSpec((1, _NCYT), lambda i: (0, 0)),
            pl.BlockSpec((1, _HID), lambda i: (0, 0)),
            pl.BlockSpec((1, _HID), lambda i: (0, 0)),
            pl.BlockSpec((1, _HID), lambda i: (0, 0)),
            pl.BlockSpec((1, 2), lambda i: (0, 0)),
        ],
        out_specs=pl.BlockSpec((_BLK, 3), lambda i: (i, 0)),
        out_shape=jax.ShapeDtypeStruct((B, 3), jnp.float32),
    )(d, fit2, M, m2, mcyt2, base2, slope2, b1r, w2a, w2b, b2r)
    return out


# speedup vs baseline: 1.8680x; 1.7664x over previous
import numpy as np
import jax
import jax.numpy as jnp
from jax.experimental import pallas as pl

_NUM_RES = 190
_CYT = np.arange(0, _NUM_RES, 4)
_NCYT = len(_CYT)
_HID = 10
_NB = 2048


def _body(d1_ref, d2_ref, f_ref, A_ref, Bm_ref, m_ref, mcyt_ref, base_ref,
          slope_ref, b1_ref, w2a_ref, w2b_ref, b2_ref, out_ref):
    d1 = d1_ref[0].astype(jnp.bfloat16)
    d2 = d2_ref[0].astype(jnp.bfloat16)
    dn = (((1,), (0,)), ((), ()))
    G32 = (jax.lax.dot_general(A_ref[...], d1, dimension_numbers=dn,
                               preferred_element_type=jnp.float32)
           + jax.lax.dot_general(Bm_ref[...], d2, dimension_numbers=dn,
                                 preferred_element_type=jnp.float32))
    G = G32[0:16] + G32[16:32]
    f = f_ref[...]

    S2 = G[0:1]
    Sxb = G[1:2]
    Sxs = G[2:3]

    h = jnp.maximum(G[3:3 + _HID] + b1_ref[...], 0.0)
    p0 = jnp.sum(h * w2a_ref[...], axis=0, keepdims=True) + b2_ref[0:1]
    p1 = jnp.sum(h * w2b_ref[...], axis=0, keepdims=True) + b2_ref[1:2]

    a = base_ref[...] + slope_ref[...] * f
    mc = mcyt_ref[...]
    mx = jnp.maximum(jnp.maximum(a, mc), 0.0)
    lse = mx + jnp.log(jnp.exp(-mx) + jnp.exp(a - mx) + jnp.exp(mc - mx))
    sum_lse = jnp.sum(lse, axis=0, keepdims=True)

    m = m_ref[...]
    r = jax.lax.broadcasted_iota(jnp.int32, (1, _NUM_RES), 1)
    sp = jnp.maximum(m, 0.0) + jnp.log1p(jnp.exp(-jnp.abs(m)))
    c_non = jnp.sum(jnp.where((r % 4) != 0, sp, 0.0))

    logp = S2 + Sxb + Sxs * f - c_non - sum_lse
    out_ref[...] = jnp.concatenate([logp, p0, p1], axis=0)


def _hilo(Mf):
    hi = Mf.astype(jnp.bfloat16)
    lo = (Mf - hi.astype(jnp.float32)).astype(jnp.bfloat16)
    return jnp.concatenate([hi, lo], axis=0)


def kernel(dharma_output, fitness, mutation_rates, baseline_edits,
           slope_edits, W1, b1, W2, b2):
    B = dharma_output.shape[0]
    dt = jnp.transpose(dharma_output, (2, 1, 0))

    A = jnp.zeros((16, _NUM_RES), dtype=jnp.float32)
    A = A.at[1, _CYT].set(baseline_edits)
    A = A.at[2, _CYT].set(slope_edits)
    A = A.at[3:3 + _HID, _CYT].set(W1.T)
    Bm = jnp.zeros((16, _NUM_RES), dtype=jnp.float32)
    Bm = Bm.at[0].set(mutation_rates)

    f2 = fitness.reshape(1, B)
    m2 = mutation_rates.reshape(1, _NUM_RES)
    mcyt = mutation_rates[_CYT].reshape(_NCYT, 1)
    base = baseline_edits.reshape(_NCYT, 1)
    slope = slope_edits.reshape(_NCYT, 1)
    b1c = b1.reshape(_HID, 1)
    w2a = W2[:, 0].reshape(_HID, 1)
    w2b = W2[:, 1].reshape(_HID, 1)
    b2c = b2.reshape(2, 1)

    grid = B // _NB
    outT = pl.pallas_call(
        _body,
        grid=(grid,),
        in_specs=[
            pl.BlockSpec((1, _NUM_RES, _NB), lambda i: (1, 0, i)),
            pl.BlockSpec((1, _NUM_RES, _NB), lambda i: (2, 0, i)),
            pl.BlockSpec((1, _NB), lambda i: (0, i)),
            pl.BlockSpec((32, _NUM_RES), lambda i: (0, 0)),
            pl.BlockSpec((32, _NUM_RES), lambda i: (0, 0)),
            pl.BlockSpec((1, _NUM_RES), lambda i: (0, 0)),
            pl.BlockSpec((_NCYT, 1), lambda i: (0, 0)),
            pl.BlockSpec((_NCYT, 1), lambda i: (0, 0)),
            pl.BlockSpec((_NCYT, 1), lambda i: (0, 0)),
            pl.BlockSpec((_HID, 1), lambda i: (0, 0)),
            pl.BlockSpec((_HID, 1), lambda i: (0, 0)),
            pl.BlockSpec((_HID, 1), lambda i: (0, 0)),
            pl.BlockSpec((2, 1), lambda i: (0, 0)),
        ],
        out_specs=pl.BlockSpec((3, _NB), lambda i: (0, i)),
        out_shape=jax.ShapeDtypeStruct((3, B), jnp.float32),
    )(dt, dt, f2, _hilo(A), _hilo(Bm), m2, mcyt, base, slope,
      b1c, w2a, w2b, b2c)
    return outT.T


# speedup vs baseline: 1.9200x; 1.0278x over previous
import numpy as np
import jax
import jax.numpy as jnp
from jax.experimental import pallas as pl

_NUM_RES = 190
_CYT = np.arange(0, _NUM_RES, 4)
_NCYT = len(_CYT)
_HID = 10
_NB = 8192


def _body(d1_ref, d2_ref, f_ref, A_ref, Bm_ref, m_ref, mcyt_ref, base_ref,
          slope_ref, b1_ref, w2a_ref, w2b_ref, b2_ref, out_ref):
    d1 = d1_ref[0].astype(jnp.bfloat16)
    d2 = d2_ref[0].astype(jnp.bfloat16)
    dn = (((1,), (0,)), ((), ()))
    G32 = (jax.lax.dot_general(A_ref[...], d1, dimension_numbers=dn,
                               preferred_element_type=jnp.float32)
           + jax.lax.dot_general(Bm_ref[...], d2, dimension_numbers=dn,
                                 preferred_element_type=jnp.float32))
    G = G32[0:16] + G32[16:32]
    f = f_ref[...]

    S2 = G[0:1]
    Sxb = G[1:2]
    Sxs = G[2:3]

    h = jnp.maximum(G[3:3 + _HID] + b1_ref[...], 0.0)
    p0 = jnp.sum(h * w2a_ref[...], axis=0, keepdims=True) + b2_ref[0:1]
    p1 = jnp.sum(h * w2b_ref[...], axis=0, keepdims=True) + b2_ref[1:2]

    a = base_ref[...] + slope_ref[...] * f
    mc = mcyt_ref[...]
    mx = jnp.maximum(jnp.maximum(a, mc), 0.0)
    lse = mx + jnp.log(jnp.exp(-mx) + jnp.exp(a - mx) + jnp.exp(mc - mx))
    sum_lse = jnp.sum(lse, axis=0, keepdims=True)

    m = m_ref[...]
    r = jax.lax.broadcasted_iota(jnp.int32, (1, _NUM_RES), 1)
    sp = jnp.maximum(m, 0.0) + jnp.log1p(jnp.exp(-jnp.abs(m)))
    c_non = jnp.sum(jnp.where((r % 4) != 0, sp, 0.0))

    logp = S2 + Sxb + Sxs * f - c_non - sum_lse
    out_ref[...] = jnp.concatenate([logp, p0, p1], axis=0)


def _hilo(Mf):
    hi = Mf.astype(jnp.bfloat16)
    lo = (Mf - hi.astype(jnp.float32)).astype(jnp.bfloat16)
    return jnp.concatenate([hi, lo], axis=0)


def kernel(dharma_output, fitness, mutation_rates, baseline_edits,
           slope_edits, W1, b1, W2, b2):
    B = dharma_output.shape[0]
    dt = jnp.transpose(dharma_output, (2, 1, 0))

    A = jnp.zeros((16, _NUM_RES), dtype=jnp.float32)
    A = A.at[1, _CYT].set(baseline_edits)
    A = A.at[2, _CYT].set(slope_edits)
    A = A.at[3:3 + _HID, _CYT].set(W1.T)
    Bm = jnp.zeros((16, _NUM_RES), dtype=jnp.float32)
    Bm = Bm.at[0].set(mutation_rates)

    f2 = fitness.reshape(1, B)
    m2 = mutation_rates.reshape(1, _NUM_RES)
    mcyt = mutation_rates[_CYT].reshape(_NCYT, 1)
    base = baseline_edits.reshape(_NCYT, 1)
    slope = slope_edits.reshape(_NCYT, 1)
    b1c = b1.reshape(_HID, 1)
    w2a = W2[:, 0].reshape(_HID, 1)
    w2b = W2[:, 1].reshape(_HID, 1)
    b2c = b2.reshape(2, 1)

    grid = B // _NB
    outT = pl.pallas_call(
        _body,
        grid=(grid,),
        in_specs=[
            pl.BlockSpec((1, _NUM_RES, _NB), lambda i: (1, 0, i)),
            pl.BlockSpec((1, _NUM_RES, _NB), lambda i: (2, 0, i)),
            pl.BlockSpec((1, _NB), lambda i: (0, i)),
            pl.BlockSpec((32, _NUM_RES), lambda i: (0, 0)),
            pl.BlockSpec((32, _NUM_RES), lambda i: (0, 0)),
            pl.BlockSpec((1, _NUM_RES), lambda i: (0, 0)),
            pl.BlockSpec((_NCYT, 1), lambda i: (0, 0)),
            pl.BlockSpec((_NCYT, 1), lambda i: (0, 0)),
            pl.BlockSpec((_NCYT, 1), lambda i: (0, 0)),
            pl.BlockSpec((_HID, 1), lambda i: (0, 0)),
            pl.BlockSpec((_HID, 1), lambda i: (0, 0)),
            pl.BlockSpec((_HID, 1), lambda i: (0, 0)),
            pl.BlockSpec((2, 1), lambda i: (0, 0)),
        ],
        out_specs=pl.BlockSpec((3, _NB), lambda i: (0, i)),
        out_shape=jax.ShapeDtypeStruct((3, B), jnp.float32),
    )(dt, dt, f2, _hilo(A), _hilo(Bm), m2, mcyt, base, slope,
      b1c, w2a, w2b, b2c)
    return outT.T
